# baseline (speedup 1.0000x reference)
import jax
import jax.numpy as jnp
from jax.experimental import pallas as pl

BLOCK_ROWS = 2048


def _body(x_ref, w0_ref, w1_ref, o_ref):
    o_ref[...] = jnp.maximum(x_ref[...] * w0_ref[...], 0.0) * w1_ref[...]


def kernel(features, w0, w1):
    n, d = features.shape
    grid = (pl.cdiv(n, BLOCK_ROWS),)
    w0r = w0.reshape(1, d)
    w1r = w1.reshape(1, d)
    return pl.pallas_call(
        _body,
        grid=grid,
        in_specs=[
            pl.BlockSpec((BLOCK_ROWS, d), lambda i: (i, 0)),
            pl.BlockSpec((1, d), lambda i: (0, 0)),
            pl.BlockSpec((1, d), lambda i: (0, 0)),
        ],
        out_specs=pl.BlockSpec((BLOCK_ROWS, d), lambda i: (i, 0)),
        out_shape=jax.ShapeDtypeStruct((n, d), features.dtype),
    )(features, w0r, w1r)
